# baseline (speedup 1.0000x reference)
import jax
import jax.numpy as jnp
from jax import lax
from jax.experimental import pallas as pl
from jax.experimental.pallas import tpu as pltpu
from jax.experimental.pallas import tpu_sc as plsc

_MAXD = 32
_NB = 2 * _MAXD + 1
_NH = 16
_N = _MAXD * _MAXD
_NWORKERS = 32
_ROWS_PER_W = (64 * 64) // _NWORKERS


def _sc_tables(w_hbm, out_hbm, idx_v, rows_v, sem):
    wid = lax.axis_index("s") * 2 + lax.axis_index("c")
    base = wid * _ROWS_PER_W
    lanes = lax.iota(jnp.int32, 16)
    for c in range(_ROWS_PER_W // 16):
        r = jnp.broadcast_to(base + 16 * c, (16,)) + lanes
        p = r >> 6
        q = r & 63
        rel = p - q + _MAXD
        in_block = (p >> 5) == (q >> 5)
        idx_v[pl.ds(16 * c, 16)] = jnp.where(in_block, rel, _NB)
    pltpu.async_copy(w_hbm.at[idx_v], rows_v, sem).wait()
    pltpu.sync_copy(rows_v, out_hbm.at[pl.ds(base, _ROWS_PER_W)])


def _tc_expand(tab_ref, o_ref):
    ablk = tab_ref[0]

    i2 = lax.broadcasted_iota(jnp.int32, (_N, 64), 0)
    p2 = lax.broadcasted_iota(jnp.int32, (_N, 64), 1)
    e_sel = jnp.where(p2 < _MAXD, i2 >> 5, i2 & 31)
    e_tgt = jnp.where(p2 < _MAXD, p2, p2 - _MAXD)
    e = jnp.where(e_sel == e_tgt, 1.0, 0.0)
    p3 = lax.broadcasted_iota(jnp.int32, (64, _N), 0)
    j3 = lax.broadcasted_iota(jnp.int32, (64, _N), 1)
    et_sel = jnp.where(p3 < _MAXD, j3 >> 5, j3 & 31)
    et_tgt = jnp.where(p3 < _MAXD, p3, p3 - _MAXD)
    et = jnp.where(et_sel == et_tgt, 1.0, 0.0)

    t = jnp.dot(ablk, et, preferred_element_type=jnp.float32)
    o_ref[0] = jnp.dot(e, t, preferred_element_type=jnp.float32)


def kernel(h, w, W):
    del h, w
    w_pad = jnp.pad(W, ((0, 1), (0, 128 - _NH)))
    mesh = plsc.VectorSubcoreMesh(core_axis_name="c", subcore_axis_name="s")
    tables = pl.kernel(
        _sc_tables,
        mesh=mesh,
        out_type=jax.ShapeDtypeStruct((64 * 64, 128), jnp.float32),
        scratch_types=[
            pltpu.VMEM((_ROWS_PER_W,), jnp.int32),
            pltpu.VMEM((_ROWS_PER_W, 128), jnp.float32),
            pltpu.SemaphoreType.DMA,
        ],
    )(w_pad)
    tabs = jnp.transpose(tables.reshape(64, 64, 128)[:, :, :_NH], (2, 0, 1))
    out = pl.pallas_call(
        _tc_expand,
        grid=(_NH,),
        in_specs=[pl.BlockSpec((1, 64, 64), lambda n: (n, 0, 0))],
        out_specs=pl.BlockSpec((1, _N, _N), lambda n: (n, 0, 0)),
        out_shape=jax.ShapeDtypeStruct((_NH, _N, _N), jnp.float32),
    )(tabs)
    return out


# speedup vs baseline: 2.4423x; 2.4423x over previous
import jax
import jax.numpy as jnp
from jax import lax
from jax.experimental import pallas as pl
from jax.experimental.pallas import tpu as pltpu
from jax.experimental.pallas import tpu_sc as plsc

_MAXD = 32
_NB = 2 * _MAXD + 1
_NH = 16
_N = _MAXD * _MAXD
_NWORKERS = 32
_ROWS_PER_W = (64 * 64) // _NWORKERS


def _sc_tables(w_hbm, out_hbm, w_v, rows_v):
    wid = lax.axis_index("s") * 2 + lax.axis_index("c")
    base = wid * _ROWS_PER_W
    pltpu.sync_copy(w_hbm, w_v.at[pl.ds(0, _NB)])
    w_v[_NB, :] = jnp.zeros((_NH,), jnp.float32)
    for m in range(_ROWS_PER_W):
        r = base + m
        p = r >> 6
        q = r & 63
        rel = p - q + _MAXD
        in_block = (p >> 5) == (q >> 5)
        idx = jnp.where(in_block, rel, _NB)
        rows_v[m, :] = w_v[idx, :]
    pltpu.sync_copy(rows_v, out_hbm.at[pl.ds(base, _ROWS_PER_W)])


def _tc_expand(tab_ref, o_ref):
    ablk = tab_ref[0]

    i2 = lax.broadcasted_iota(jnp.int32, (_N, 64), 0)
    p2 = lax.broadcasted_iota(jnp.int32, (_N, 64), 1)
    e_sel = jnp.where(p2 < _MAXD, i2 >> 5, i2 & 31)
    e_tgt = jnp.where(p2 < _MAXD, p2, p2 - _MAXD)
    e = jnp.where(e_sel == e_tgt, 1.0, 0.0)
    p3 = lax.broadcasted_iota(jnp.int32, (64, _N), 0)
    j3 = lax.broadcasted_iota(jnp.int32, (64, _N), 1)
    et_sel = jnp.where(p3 < _MAXD, j3 >> 5, j3 & 31)
    et_tgt = jnp.where(p3 < _MAXD, p3, p3 - _MAXD)
    et = jnp.where(et_sel == et_tgt, 1.0, 0.0)

    t = jnp.dot(ablk, et, preferred_element_type=jnp.float32)
    o_ref[0] = jnp.dot(e, t, preferred_element_type=jnp.float32)


def kernel(h, w, W):
    del h, w
    mesh = plsc.VectorSubcoreMesh(core_axis_name="c", subcore_axis_name="s")
    tables = pl.kernel(
        _sc_tables,
        mesh=mesh,
        out_type=jax.ShapeDtypeStruct((64 * 64, _NH), jnp.float32),
        scratch_types=[
            pltpu.VMEM((_NB + 1, _NH), jnp.float32),
            pltpu.VMEM((_ROWS_PER_W, _NH), jnp.float32),
        ],
    )(W)
    tabs = jnp.transpose(tables.reshape(64, 64, _NH), (2, 0, 1))
    out = pl.pallas_call(
        _tc_expand,
        grid=(_NH,),
        in_specs=[pl.BlockSpec((1, 64, 64), lambda n: (n, 0, 0))],
        out_specs=pl.BlockSpec((1, _N, _N), lambda n: (n, 0, 0)),
        out_shape=jax.ShapeDtypeStruct((_NH, _N, _N), jnp.float32),
    )(tabs)
    return out
